# baseline (speedup 1.0000x reference)
import functools

import jax
import jax.numpy as jnp
from jax import lax
from jax.experimental import pallas as pl
from jax.experimental.pallas import tpu as pltpu
from jax.experimental.pallas import tpu_sc as plsc

_VOCAB = 1000000
_DIM = 64
_ROWS = 4096
_COLS = 200
_B = _ROWS * _COLS

_NC = 2
_NS = 16
_NW = _NC * _NS
_BPW = _B // _NW

_IW = 128
_NR = _BPW // _IW
_K = 4
_C = _IW * _K
_NCHUNK = _BPW // _C
_NG = _NCHUNK // 2

_mesh = plsc.VectorSubcoreMesh(core_axis_name="c", subcore_axis_name="s")


@functools.partial(
    pl.kernel,
    mesh=_mesh,
    out_type=jax.ShapeDtypeStruct((_B, _DIM), jnp.float32),
    compiler_params=pltpu.CompilerParams(use_tc_tiling_on_sc=False),
    scratch_types=[
        pltpu.VMEM((_NCHUNK, _C), jnp.int32),
        pltpu.VMEM((2, _C, _DIM), jnp.float32),
        pltpu.SemaphoreType.DMA,
        pltpu.SemaphoreType.DMA,
        pltpu.SemaphoreType.DMA,
        pltpu.SemaphoreType.DMA,
    ],
)
def _embed_sc(x_hbm, table_hbm, out_hbm, idx_v, rows_v, g0, g1, o0, o1):
    wid = lax.axis_index("s") * _NC + lax.axis_index("c")
    base = wid * _BPW
    gsems = (g0, g1)
    osems = (o0, o1)

    pltpu.sync_copy(x_hbm.at[pl.ds(wid * _NCHUNK, _NCHUNK)], idx_v)

    def fire_gather(i, b):
        pltpu.async_copy(
            table_hbm.at[idx_v.at[i]],
            rows_v.at[b],
            gsems[b],
        )

    def wait_gather(b):
        pltpu.make_async_copy(
            table_hbm.at[pl.ds(0, _C)], rows_v.at[b], gsems[b]
        ).wait()

    def fire_out(i, b):
        del i
        pltpu.async_copy(rows_v.at[b], out_hbm.at[pl.ds(base, _C)], osems[b])

    def wait_out(b):
        pltpu.make_async_copy(
            rows_v.at[b], out_hbm.at[pl.ds(base, _C)], osems[b]
        ).wait()

    fire_gather(0, 0)

    def group(g, carry):
        for b in range(2):
            i = 2 * g + b
            nb = 1 - b

            @pl.when(i + 1 < _NCHUNK)
            def _fire_next():
                @pl.when(i >= 1)
                def _free_buffer():
                    wait_out(nb)

                fire_gather(i + 1, nb)

            wait_gather(b)
            fire_out(i, b)
        return carry

    lax.fori_loop(0, _NG, group, 0)
    wait_out(0)
    wait_out(1)


def kernel(x, table):
    xf = jnp.asarray(x, jnp.int32).reshape(_B // _C, _C)
    out = _embed_sc(xf, table)
    return out.reshape(_ROWS, _COLS, _DIM)


# speedup vs baseline: 1.0205x; 1.0205x over previous
import functools

import jax
import jax.numpy as jnp
from jax import lax
from jax.experimental import pallas as pl
from jax.experimental.pallas import tpu as pltpu
from jax.experimental.pallas import tpu_sc as plsc

_VOCAB = 1000000
_DIM = 64
_ROWS = 4096
_COLS = 200
_B = _ROWS * _COLS

_NC = 2
_NS = 16
_NW = _NC * _NS
_BPW = _B // _NW

_IW = 128
_NR = _BPW // _IW
_K = 4
_C = _IW * _K
_NCHUNK = _BPW // _C
_NG = _NCHUNK // 2

_mesh = plsc.VectorSubcoreMesh(core_axis_name="c", subcore_axis_name="s")


@functools.partial(
    pl.kernel,
    mesh=_mesh,
    out_type=jax.ShapeDtypeStruct((_B, _DIM), jnp.float32),
    compiler_params=pltpu.CompilerParams(use_tc_tiling_on_sc=False),
    scratch_types=[
        pltpu.VMEM((_NCHUNK, _C), jnp.int32),
        pltpu.VMEM((2, _C, _DIM), jnp.float32),
        pltpu.SemaphoreType.DMA,
        pltpu.SemaphoreType.DMA,
        pltpu.SemaphoreType.DMA,
        pltpu.SemaphoreType.DMA,
    ],
)
def _embed_sc(x_hbm, table_hbm, out_hbm, idx_v, rows_v, g0, g1, o0, o1):
    wid = lax.axis_index("s") * _NC + lax.axis_index("c")
    base = wid * _BPW
    gsems = (g0, g1)
    osems = (o0, o1)

    pltpu.sync_copy(x_hbm.at[pl.ds(wid * _NCHUNK, _NCHUNK)], idx_v)

    def fire_gather(i, b):
        pltpu.async_copy(
            table_hbm.at[idx_v.at[i]],
            rows_v.at[b],
            gsems[b],
        )

    def wait_gather(b):
        pltpu.make_async_copy(
            table_hbm.at[pl.ds(0, _C)], rows_v.at[b], gsems[b]
        ).wait()

    def fire_out(i, b):
        pltpu.async_copy(rows_v.at[b], out_hbm.at[pl.ds(base + i * _C, _C)], osems[b])

    def wait_out(b):
        pltpu.make_async_copy(
            rows_v.at[b], out_hbm.at[pl.ds(base, _C)], osems[b]
        ).wait()

    fire_gather(0, 0)

    def group(g, carry):
        for b in range(2):
            i = 2 * g + b
            nb = 1 - b

            @pl.when(i + 1 < _NCHUNK)
            def _fire_next():
                @pl.when(i >= 1)
                def _free_buffer():
                    wait_out(nb)

                fire_gather(i + 1, nb)

            wait_gather(b)
            fire_out(i, b)
        return carry

    lax.fori_loop(0, _NG, group, 0)
    wait_out(0)
    wait_out(1)


def kernel(x, table):
    xf = jnp.asarray(x, jnp.int32).reshape(_B // _C, _C)
    out = _embed_sc(xf, table)
    return out.reshape(_ROWS, _COLS, _DIM)
